# baseline (speedup 1.0000x reference)
import functools

import jax
import jax.numpy as jnp
from jax import lax
from jax.experimental import pallas as pl
from jax.experimental.pallas import tpu as pltpu
from jax.experimental.pallas import tpu_sc as plsc

NUM_EMB = 100000
D_BYTES = 128
D_WORDS = D_BYTES // 4
B_TOTAL = 4096 * 26
NC, NS, L = 2, 16, 16
NW = NC * NS
ROWS_PER_W = B_TOTAL // NW
CHUNK = 128
CHUNKS_PER_W = ROWS_PER_W // CHUNK


def _body(qw_hbm, idx_hbm, scale_hbm, bias_hbm, out_hbm,
          idxc_v, rows_v, out_v, sv_v, bv_v, sem):
    wid = lax.axis_index("s") * NC + lax.axis_index("c")

    pltpu.sync_copy(scale_hbm, sv_v)
    pltpu.sync_copy(bias_hbm, bv_v)
    scale = sv_v[...]
    bias = bv_v[...]

    lanes = lax.iota(jnp.int32, L)
    col_base = [[64 * w + 4 * lanes + j for j in range(4)] for w in range(2)]

    def do_chunk(c, _):
        pltpu.sync_copy(idx_hbm.at[wid * CHUNKS_PER_W + c], idxc_v)
        pltpu.async_copy(qw_hbm.at[idxc_v], rows_v, sem).wait()

        def do_row(r, _):
            rowoff = r * D_BYTES
            for w in range(2):
                x = rows_v[r, pl.ds(16 * w, 16)]
                for j in range(4):
                    sh = 8 * (3 - j)
                    t = lax.shift_right_arithmetic(
                        lax.shift_left(x, sh) if sh else x, 24)
                    f = t.astype(jnp.float32) * scale + bias
                    plsc.store_scatter(out_v, [rowoff + col_base[w][j]], f)
            return 0

        lax.fori_loop(0, CHUNK, do_row, 0)
        pltpu.sync_copy(
            out_v,
            out_hbm.at[pl.ds((wid * CHUNKS_PER_W + c) * CHUNK * D_BYTES,
                             CHUNK * D_BYTES)])
        return 0

    lax.fori_loop(0, CHUNKS_PER_W, do_chunk, 0)


@jax.jit
def _run(qw32, idx2d, scale_vec, bias_vec):
    mesh = plsc.VectorSubcoreMesh(
        core_axis_name="c", subcore_axis_name="s",
        num_cores=NC, num_subcores=NS)
    return pl.kernel(
        _body,
        out_type=jax.ShapeDtypeStruct((B_TOTAL * D_BYTES,), jnp.float32),
        mesh=mesh,
        scratch_types=[
            pltpu.VMEM((CHUNK,), jnp.int32),
            pltpu.VMEM((CHUNK, D_WORDS), jnp.int32),
            pltpu.VMEM((CHUNK * D_BYTES,), jnp.float32),
            pltpu.VMEM((L,), jnp.float32),
            pltpu.VMEM((L,), jnp.float32),
            pltpu.SemaphoreType.DMA,
        ],
        compiler_params=pltpu.CompilerParams(
            needs_layout_passes=False, use_tc_tiling_on_sc=False),
    )(qw32, idx2d, scale_vec, bias_vec)


def kernel(input, qweight, scale, zero_point):
    qw32 = lax.bitcast_convert_type(
        qweight.reshape(NUM_EMB, D_WORDS, 4), jnp.int32)
    idx2d = input.astype(jnp.int32).reshape(B_TOTAL // CHUNK, CHUNK)
    scale_f = scale.astype(jnp.float32)
    bias_f = -zero_point.astype(jnp.float32) * scale_f
    scale_vec = jnp.full((L,), scale_f, jnp.float32)
    bias_vec = jnp.full((L,), bias_f, jnp.float32)
    flat = _run(qw32, idx2d, scale_vec, bias_vec)
    return flat.reshape(input.shape[0], input.shape[1], D_BYTES)


# speedup vs baseline: 2.6876x; 2.6876x over previous
import functools

import jax
import jax.numpy as jnp
from jax import lax
from jax.experimental import pallas as pl
from jax.experimental.pallas import tpu as pltpu
from jax.experimental.pallas import tpu_sc as plsc

NUM_EMB = 100000
D_BYTES = 128
B_TOTAL = 4096 * 26
NC, NS, L = 2, 16, 16
NW = NC * NS
ROWS_PER_W = B_TOTAL // NW
CHUNK = 128
CHUNKS_PER_W = ROWS_PER_W // CHUNK


def _body(qw_hbm, idx_hbm, scale_hbm, bias_hbm, out_hbm,
          idxc_v, rows_v, out_v, sv_v, bv_v, sem):
    wid = lax.axis_index("s") * NC + lax.axis_index("c")

    pltpu.sync_copy(scale_hbm, sv_v)
    pltpu.sync_copy(bias_hbm, bv_v)
    scale = sv_v[...]
    bias = bv_v[...]

    lanes = lax.iota(jnp.int32, L)
    col_base = [[64 * w + 4 * lanes + j for j in range(4)] for w in range(2)]

    def do_chunk(c, _):
        pltpu.sync_copy(idx_hbm.at[wid * CHUNKS_PER_W + c], idxc_v)
        pltpu.async_copy(qw_hbm.at[idxc_v], rows_v, sem).wait()

        def do_row(r, _):
            rowoff = r * D_BYTES
            for w in range(2):
                x8 = rows_v[r, pl.ds(64 * w, 64)]
                x = plsc.bitcast(x8, jnp.int32)
                for j in range(4):
                    sh = 8 * (3 - j)
                    t = lax.shift_right_arithmetic(
                        lax.shift_left(x, sh) if sh else x, 24)
                    f = t.astype(jnp.float32) * scale + bias
                    plsc.store_scatter(out_v, [rowoff + col_base[w][j]], f)
            return 0

        lax.fori_loop(0, CHUNK, do_row, 0)
        pltpu.sync_copy(
            out_v,
            out_hbm.at[pl.ds((wid * CHUNKS_PER_W + c) * CHUNK * D_BYTES,
                             CHUNK * D_BYTES)])
        return 0

    lax.fori_loop(0, CHUNKS_PER_W, do_chunk, 0)


@jax.jit
def _run(qweight, idx2d, scale_vec, bias_vec):
    mesh = plsc.VectorSubcoreMesh(
        core_axis_name="c", subcore_axis_name="s",
        num_cores=NC, num_subcores=NS)
    return pl.kernel(
        _body,
        out_type=jax.ShapeDtypeStruct((B_TOTAL * D_BYTES,), jnp.float32),
        mesh=mesh,
        scratch_types=[
            pltpu.VMEM((CHUNK,), jnp.int32),
            pltpu.VMEM((CHUNK, D_BYTES), jnp.int8),
            pltpu.VMEM((CHUNK * D_BYTES,), jnp.float32),
            pltpu.VMEM((L,), jnp.float32),
            pltpu.VMEM((L,), jnp.float32),
            pltpu.SemaphoreType.DMA,
        ],
        compiler_params=pltpu.CompilerParams(
            needs_layout_passes=False, use_tc_tiling_on_sc=False),
    )(qweight, idx2d, scale_vec, bias_vec)


def kernel(input, qweight, scale, zero_point):
    nb, nf = input.shape
    idxT = jnp.swapaxes(input.astype(jnp.int32), 0, 1)
    idx2d = idxT.reshape(B_TOTAL // CHUNK, CHUNK)
    scale_f = scale.astype(jnp.float32)
    bias_f = -zero_point.astype(jnp.float32) * scale_f
    scale_vec = jnp.full((L,), scale_f, jnp.float32)
    bias_vec = jnp.full((L,), bias_f, jnp.float32)
    flat = _run(qweight, idx2d, scale_vec, bias_vec)
    return jnp.transpose(flat.reshape(nf, nb, D_BYTES), (1, 0, 2))


# speedup vs baseline: 3.3807x; 1.2579x over previous
import functools

import jax
import jax.numpy as jnp
from jax import lax
from jax.experimental import pallas as pl
from jax.experimental.pallas import tpu as pltpu
from jax.experimental.pallas import tpu_sc as plsc

NUM_EMB = 100000
D_BYTES = 128
B_TOTAL = 4096 * 26
NC, NS, L = 2, 16, 16
NW = NC * NS
ROWS_PER_W = B_TOTAL // NW
CHUNK = 128
CHUNKS_PER_W = ROWS_PER_W // CHUNK


def _body(qw_hbm, idx_hbm, scale_hbm, bias_hbm, out_hbm,
          idx_v, rows0_v, rows1_v, out0_v, out1_v, sv_v, bv_v,
          gsem0, gsem1, osem0, osem1):
    wid = lax.axis_index("s") * NC + lax.axis_index("c")

    pltpu.sync_copy(scale_hbm, sv_v)
    pltpu.sync_copy(bias_hbm, bv_v)
    pltpu.sync_copy(idx_hbm.at[pl.ds(wid * CHUNKS_PER_W, CHUNKS_PER_W)], idx_v)
    scale = sv_v[...]
    bias = bv_v[...]

    lanes = lax.iota(jnp.int32, L)
    col_base = [[64 * w + 4 * lanes + j for j in range(4)] for w in range(2)]

    rows = (rows0_v, rows1_v)
    outs = (out0_v, out1_v)
    gsems = (gsem0, gsem1)
    osems = (osem0, osem1)

    def start_gather(c):
        b = c & 1
        return pltpu.async_copy(qw_hbm.at[idx_v.at[c]], rows[b], gsems[b])

    def dequant(c):
        b = c & 1
        rows_b, out_b = rows[b], outs[b]

        def do_row(r, _):
            rowoff = r * D_BYTES
            for w in range(2):
                x8 = rows_b[r, pl.ds(64 * w, 64)]
                x = plsc.bitcast(x8, jnp.int32)
                for j in range(4):
                    sh = 8 * (3 - j)
                    t = lax.shift_right_arithmetic(
                        lax.shift_left(x, sh) if sh else x, 24)
                    f = t.astype(jnp.float32) * scale + bias
                    plsc.store_scatter(out_b, [rowoff + col_base[w][j]], f)
            return 0

        lax.fori_loop(0, CHUNK, do_row, 0)

    def start_writeback(c):
        b = c & 1
        dst = out_hbm.at[pl.ds((wid * CHUNKS_PER_W + c) * CHUNK * D_BYTES,
                               CHUNK * D_BYTES)]
        return pltpu.async_copy(outs[b], dst, osems[b])

    gh = [None] * CHUNKS_PER_W
    oh = [None] * CHUNKS_PER_W
    gh[0] = start_gather(0)
    for c in range(CHUNKS_PER_W):
        if c + 1 < CHUNKS_PER_W:
            gh[c + 1] = start_gather(c + 1)
        gh[c].wait()
        if c >= 2:
            oh[c - 2].wait()
        dequant(c)
        oh[c] = start_writeback(c)
    oh[CHUNKS_PER_W - 2].wait()
    oh[CHUNKS_PER_W - 1].wait()


@jax.jit
def _run(qweight, idx2d, scale_vec, bias_vec):
    mesh = plsc.VectorSubcoreMesh(
        core_axis_name="c", subcore_axis_name="s",
        num_cores=NC, num_subcores=NS)
    return pl.kernel(
        _body,
        out_type=jax.ShapeDtypeStruct((B_TOTAL * D_BYTES,), jnp.float32),
        mesh=mesh,
        scratch_types=[
            pltpu.VMEM((CHUNKS_PER_W, CHUNK), jnp.int32),
            pltpu.VMEM((CHUNK, D_BYTES), jnp.int8),
            pltpu.VMEM((CHUNK, D_BYTES), jnp.int8),
            pltpu.VMEM((CHUNK * D_BYTES,), jnp.float32),
            pltpu.VMEM((CHUNK * D_BYTES,), jnp.float32),
            pltpu.VMEM((L,), jnp.float32),
            pltpu.VMEM((L,), jnp.float32),
            pltpu.SemaphoreType.DMA,
            pltpu.SemaphoreType.DMA,
            pltpu.SemaphoreType.DMA,
            pltpu.SemaphoreType.DMA,
        ],
        compiler_params=pltpu.CompilerParams(
            needs_layout_passes=False, use_tc_tiling_on_sc=False),
    )(qweight, idx2d, scale_vec, bias_vec)


def kernel(input, qweight, scale, zero_point):
    nb, nf = input.shape
    idxT = jnp.swapaxes(input.astype(jnp.int32), 0, 1)
    idx2d = idxT.reshape(B_TOTAL // CHUNK, CHUNK)
    scale_f = scale.astype(jnp.float32)
    bias_f = -zero_point.astype(jnp.float32) * scale_f
    scale_vec = jnp.full((L,), scale_f, jnp.float32)
    bias_vec = jnp.full((L,), bias_f, jnp.float32)
    flat = _run(qweight, idx2d, scale_vec, bias_vec)
    return jnp.transpose(flat.reshape(nf, nb, D_BYTES), (1, 0, 2))
